# Initial kernel scaffold; baseline (speedup 1.0000x reference)
import jax
import jax.numpy as jnp
from jax.experimental import pallas as pl


def kernel(z, W1, b1, W2, b2, Wg, att_src, att_dst, bias_g, edge_index):
    raise NotImplementedError("write your pallas kernel here")


# speedup vs baseline: 20.5317x; 20.5317x over previous
import functools

import jax
import jax.numpy as jnp
from jax import lax
from jax.experimental import pallas as pl
from jax.experimental.pallas import tpu as pltpu, tpu_sc as plsc

HEADS = 4
OUT_DIM = 128
NEG_SLOPE = 0.2
BLK = 1000

N_NODES = 10000
N_PAD = 10240
EPT = 128
NCH = 162
E_PAD = 16 * NCH * EPT

_NC, _NS, _L = 2, 16, 16


def _dense_body(z_ref, W1_ref, b1_ref, W2_ref, b2_ref, Wg_ref, asv_ref, adv_ref,
                h_ref, asrc_ref, adst_ref):
    z = z_ref[...]
    x = jnp.maximum(z @ W1_ref[...] + b1_ref[...], 0.0)
    x = jnp.maximum(x @ W2_ref[...] + b2_ref[...], 0.0)
    h = x @ Wg_ref[...]
    h_ref[...] = h
    hh = h.reshape(h.shape[0], HEADS, OUT_DIM)
    asrc_ref[...] = jnp.sum(hh * asv_ref[...], axis=-1)
    adst_ref[...] = jnp.sum(hh * adv_ref[...], axis=-1)


def _dense_stage(z, W1, b1, W2, b2, Wg, att_src, att_dst):
    N = z.shape[0]
    grid = (N // BLK,)
    out_shapes = (
        jax.ShapeDtypeStruct((N, HEADS * OUT_DIM), jnp.float32),
        jax.ShapeDtypeStruct((N, HEADS), jnp.float32),
        jax.ShapeDtypeStruct((N, HEADS), jnp.float32),
    )
    full = lambda shape: pl.BlockSpec(shape, lambda i: tuple(0 for _ in shape))
    return pl.pallas_call(
        _dense_body,
        grid=grid,
        in_specs=[
            pl.BlockSpec((BLK, 128), lambda i: (i, 0)),
            full((128, 64)), full((1, 64)), full((64, 128)), full((1, 128)),
            full((128, HEADS * OUT_DIM)),
            full((1, HEADS, OUT_DIM)), full((1, HEADS, OUT_DIM)),
        ],
        out_specs=(
            pl.BlockSpec((BLK, HEADS * OUT_DIM), lambda i: (i, 0)),
            pl.BlockSpec((BLK, HEADS), lambda i: (i, 0)),
            pl.BlockSpec((BLK, HEADS), lambda i: (i, 0)),
        ),
        out_shape=out_shapes,
    )(z, W1, b1.reshape(1, -1), W2, b2.reshape(1, -1), Wg, att_src, att_dst)


_sc_mesh = plsc.VectorSubcoreMesh(core_axis_name="c", subcore_axis_name="s")


@functools.partial(
    pl.kernel, mesh=_sc_mesh,
    out_type=(
        jax.ShapeDtypeStruct((HEADS, N_PAD, OUT_DIM), jnp.float32),
        jax.ShapeDtypeStruct((HEADS, N_PAD), jnp.float32),
    ),
    scratch_types=[
        pltpu.VMEM((EPT,), jnp.int32),
        pltpu.VMEM((EPT,), jnp.int32),
        pltpu.VMEM((EPT,), jnp.int32),
        pltpu.VMEM((EPT,), jnp.int32),
        pltpu.VMEM((EPT,), jnp.float32),
        pltpu.VMEM((EPT,), jnp.float32),
        pltpu.VMEM((EPT,), jnp.float32),
        pltpu.VMEM((EPT, OUT_DIM), jnp.float32),
        pltpu.VMEM((16, OUT_DIM), jnp.float32),
        pltpu.VMEM((N_PAD // _NS,), jnp.float32),
        pltpu.VMEM_SHARED((N_PAD, OUT_DIM), jnp.float32),
        pltpu.VMEM_SHARED((N_PAD,), jnp.float32),
        pltpu.SemaphoreType.DMA,
        pltpu.SemaphoreType.DMA,
        pltpu.SemaphoreType.DMA,
    ],
)
def _edge_kernel(hflat_hbm, asrc_hbm, adst_hbm, srcs_hbm, dsts_hbm,
                 acc_out, dacc_out,
                 sidx_v, didx_v, gsidx_v, gdidx_v, asv_v, adv_v, w_v, rows_v,
                 zb_v, zd_v, acc_sh, dacc_sh, sem, sem2, sem3):
    c = lax.axis_index("c")
    s = lax.axis_index("s")
    tile_rows = N_PAD // _NS

    z16 = jnp.zeros((_L,), jnp.float32)
    for r in range(16):
        for j in range(OUT_DIM // _L):
            zb_v[r, pl.ds(j * _L, _L)] = z16
    for j in range(tile_rows // _L):
        zd_v[pl.ds(j * _L, _L)] = z16

    for hp in range(2):
        head = c * 2 + hp
        for r8 in range(tile_rows // 16):
            pltpu.sync_copy(zb_v, acc_sh.at[pl.ds(s * tile_rows + r8 * 16, 16)])
        pltpu.sync_copy(zd_v, dacc_sh.at[pl.ds(s * tile_rows, tile_rows)])
        plsc.subcore_barrier()

        off = head * N_PAD

        def chunk_body(i, carry):
            base = (s * NCH + i) * EPT
            pltpu.sync_copy(srcs_hbm.at[pl.ds(base, EPT)], sidx_v)
            pltpu.sync_copy(dsts_hbm.at[pl.ds(base, EPT)], didx_v)
            for g in range(EPT // _L):
                sl = pl.ds(g * _L, _L)
                gsidx_v[sl] = sidx_v[sl] + off
                gdidx_v[sl] = didx_v[sl] + off
            cp_rows = pltpu.async_copy(hflat_hbm.at[gsidx_v], rows_v, sem)
            pltpu.async_copy(asrc_hbm.at[gsidx_v], asv_v, sem2).wait()
            pltpu.async_copy(adst_hbm.at[gdidx_v], adv_v, sem3).wait()
            for g in range(EPT // _L):
                sl = pl.ds(g * _L, _L)
                al = asv_v[sl] + adv_v[sl]
                al = jnp.where(al > 0.0, al, NEG_SLOPE * al)
                w_v[sl] = jnp.exp(jnp.minimum(al, 50.0))
            cp_rows.wait()
            for g in range(EPT // _L):
                wch = w_v[pl.ds(g * _L, _L)]
                for e in range(_L):
                    wv = jnp.full((_L,), wch[e])
                    r = g * _L + e
                    for j in range(OUT_DIM // _L):
                        sl = pl.ds(j * _L, _L)
                        rows_v[r, sl] = rows_v[r, sl] * wv
            pltpu.sync_copy(rows_v, acc_sh.at[didx_v], add=True)
            pltpu.sync_copy(w_v, dacc_sh.at[didx_v], add=True)
            return carry

        lax.fori_loop(0, NCH, chunk_body, 0)
        plsc.subcore_barrier()

        pltpu.sync_copy(acc_sh.at[pl.ds(s * tile_rows, tile_rows)],
                        acc_out.at[head, pl.ds(s * tile_rows, tile_rows)])
        pltpu.sync_copy(dacc_sh.at[pl.ds(s * tile_rows, tile_rows)],
                        dacc_out.at[head, pl.ds(s * tile_rows, tile_rows)])


def _epi_body(acc_ref, dacc_ref, bias_ref, out_ref):
    a = acc_ref[...]
    d = dacc_ref[...]
    r = a / (d + 1e-16)
    out_ref[...] = jnp.mean(r, axis=0) + bias_ref[...]


def _epilogue(acc, dacc, bias_g):
    grid = (N_NODES // BLK,)
    return pl.pallas_call(
        _epi_body,
        grid=grid,
        in_specs=[
            pl.BlockSpec((HEADS, BLK, OUT_DIM), lambda i: (0, i, 0)),
            pl.BlockSpec((HEADS, BLK, 1), lambda i: (0, i, 0)),
            pl.BlockSpec((1, OUT_DIM), lambda i: (0, 0)),
        ],
        out_specs=pl.BlockSpec((BLK, OUT_DIM), lambda i: (i, 0)),
        out_shape=jax.ShapeDtypeStruct((N_NODES, OUT_DIM), jnp.float32),
    )(acc, dacc, bias_g.reshape(1, -1))


def kernel(z, W1, b1, W2, b2, Wg, att_src, att_dst, bias_g, edge_index):
    N = z.shape[0]
    h_flat, a_src, a_dst = _dense_stage(z, W1, b1, W2, b2, Wg, att_src, att_dst)

    h_heads = jnp.pad(h_flat.reshape(N, HEADS, OUT_DIM).transpose(1, 0, 2),
                      ((0, 0), (0, N_PAD - N), (0, 0)))
    hflat = h_heads.reshape(HEADS * N_PAD, OUT_DIM)
    asrc_f = jnp.pad(a_src.T, ((0, 0), (0, N_PAD - N))).reshape(HEADS * N_PAD)
    adst_f = jnp.pad(a_dst.T, ((0, 0), (0, N_PAD - N))).reshape(HEADS * N_PAD)

    loops = jnp.arange(N, dtype=edge_index.dtype)
    npad_e = E_PAD - edge_index.shape[1] - N
    pad_idx = N + 16 + (jnp.arange(npad_e, dtype=jnp.int32) % 128)
    srcs = jnp.concatenate([edge_index[0], loops, pad_idx])
    dsts = jnp.concatenate([edge_index[1], loops, pad_idx])

    acc, dacc = _edge_kernel(hflat, asrc_f, adst_f, srcs, dsts)
    return _epilogue(acc[:, :N, :], dacc[:, :N].reshape(HEADS, N, 1), bias_g)


# speedup vs baseline: 32.9023x; 1.6025x over previous
import functools

import jax
import jax.numpy as jnp
from jax import lax
from jax.experimental import pallas as pl
from jax.experimental.pallas import tpu as pltpu, tpu_sc as plsc

HEADS = 4
OUT_DIM = 128
NEG_SLOPE = 0.2
BLK = 1000

N_NODES = 10000
N_PAD = 10240
EPT = 128
NCH = 162
E_PAD = 16 * NCH * EPT

_NC, _NS, _L = 2, 16, 16


def _dense_body(z_ref, W1_ref, b1_ref, W2_ref, b2_ref, Wg_ref, asv_ref, adv_ref,
                h_ref, asrc_ref, adst_ref):
    z = z_ref[...]
    x = jnp.maximum(z @ W1_ref[...] + b1_ref[...], 0.0)
    x = jnp.maximum(x @ W2_ref[...] + b2_ref[...], 0.0)
    h = x @ Wg_ref[...]
    h_ref[...] = h
    hh = h.reshape(h.shape[0], HEADS, OUT_DIM)
    asrc_ref[...] = jnp.sum(hh * asv_ref[...], axis=-1)
    adst_ref[...] = jnp.sum(hh * adv_ref[...], axis=-1)


def _dense_stage(z, W1, b1, W2, b2, Wg, att_src, att_dst):
    N = z.shape[0]
    grid = (N // BLK,)
    out_shapes = (
        jax.ShapeDtypeStruct((N, HEADS * OUT_DIM), jnp.float32),
        jax.ShapeDtypeStruct((N, HEADS), jnp.float32),
        jax.ShapeDtypeStruct((N, HEADS), jnp.float32),
    )
    full = lambda shape: pl.BlockSpec(shape, lambda i: tuple(0 for _ in shape))
    return pl.pallas_call(
        _dense_body,
        grid=grid,
        in_specs=[
            pl.BlockSpec((BLK, 128), lambda i: (i, 0)),
            full((128, 64)), full((1, 64)), full((64, 128)), full((1, 128)),
            full((128, HEADS * OUT_DIM)),
            full((1, HEADS, OUT_DIM)), full((1, HEADS, OUT_DIM)),
        ],
        out_specs=(
            pl.BlockSpec((BLK, HEADS * OUT_DIM), lambda i: (i, 0)),
            pl.BlockSpec((BLK, HEADS), lambda i: (i, 0)),
            pl.BlockSpec((BLK, HEADS), lambda i: (i, 0)),
        ),
        out_shape=out_shapes,
    )(z, W1, b1.reshape(1, -1), W2, b2.reshape(1, -1), Wg, att_src, att_dst)


_sc_mesh = plsc.VectorSubcoreMesh(core_axis_name="c", subcore_axis_name="s")


@functools.partial(
    pl.kernel, mesh=_sc_mesh,
    out_type=(
        jax.ShapeDtypeStruct((HEADS, N_PAD, OUT_DIM), jnp.float32),
        jax.ShapeDtypeStruct((HEADS, N_PAD), jnp.float32),
    ),
    scratch_types=[
        pltpu.VMEM((2, EPT), jnp.int32),
        pltpu.VMEM((2, EPT), jnp.int32),
        pltpu.VMEM((2, EPT), jnp.int32),
        pltpu.VMEM((2, EPT), jnp.int32),
        pltpu.VMEM((2, EPT), jnp.float32),
        pltpu.VMEM((2, EPT), jnp.float32),
        pltpu.VMEM((2, EPT), jnp.float32),
        pltpu.VMEM((2, EPT, OUT_DIM), jnp.float32),
        pltpu.VMEM((16, OUT_DIM), jnp.float32),
        pltpu.VMEM((N_PAD // _NS,), jnp.float32),
        pltpu.VMEM_SHARED((N_PAD, OUT_DIM), jnp.float32),
        pltpu.VMEM_SHARED((N_PAD,), jnp.float32),
        pltpu.SemaphoreType.DMA,
        pltpu.SemaphoreType.DMA,
        pltpu.SemaphoreType.DMA,
        pltpu.SemaphoreType.DMA,
        pltpu.SemaphoreType.DMA,
        pltpu.SemaphoreType.DMA,
    ],
)
def _edge_kernel(hflat_hbm, asrc_hbm, adst_hbm, srcs_hbm, dsts_hbm,
                 acc_out, dacc_out,
                 sidx_v, didx_v, gsidx_v, gdidx_v, asv_v, adv_v, w_v, rows_v,
                 zb_v, zd_v, acc_sh, dacc_sh,
                 semr0, semr1, sema0, sema1, semb0, semb1):
    c = lax.axis_index("c")
    s = lax.axis_index("s")
    tile_rows = N_PAD // _NS
    semr = (semr0, semr1)
    sema = (sema0, sema1)
    semb = (semb0, semb1)

    z16 = jnp.zeros((_L,), jnp.float32)
    for r in range(16):
        for j in range(OUT_DIM // _L):
            zb_v[r, pl.ds(j * _L, _L)] = z16
    for j in range(tile_rows // _L):
        zd_v[pl.ds(j * _L, _L)] = z16

    for hp in range(2):
        head = c * 2 + hp
        for r8 in range(tile_rows // 16):
            pltpu.sync_copy(zb_v, acc_sh.at[pl.ds(s * tile_rows + r8 * 16, 16)])
        pltpu.sync_copy(zd_v, dacc_sh.at[pl.ds(s * tile_rows, tile_rows)])
        plsc.subcore_barrier()

        off = head * N_PAD

        def issue_loads(i, p):
            base = (s * NCH + i) * EPT
            pltpu.sync_copy(srcs_hbm.at[pl.ds(base, EPT)], sidx_v.at[p])
            pltpu.sync_copy(dsts_hbm.at[pl.ds(base, EPT)], didx_v.at[p])
            for g in range(EPT // _L):
                sl = pl.ds(g * _L, _L)
                gsidx_v[p, sl] = sidx_v[p, sl] + off
                gdidx_v[p, sl] = didx_v[p, sl] + off
            pltpu.async_copy(hflat_hbm.at[gsidx_v.at[p]], rows_v.at[p], semr[p])
            pltpu.async_copy(asrc_hbm.at[gsidx_v.at[p]], asv_v.at[p], sema[p])
            pltpu.async_copy(adst_hbm.at[gdidx_v.at[p]], adv_v.at[p], semb[p])

        def compute_and_scatter(p):
            pltpu.make_async_copy(asrc_hbm.at[gsidx_v.at[p]], asv_v.at[p],
                                  sema[p]).wait()
            pltpu.make_async_copy(adst_hbm.at[gdidx_v.at[p]], adv_v.at[p],
                                  semb[p]).wait()
            for g in range(EPT // _L):
                sl = pl.ds(g * _L, _L)
                al = asv_v[p, sl] + adv_v[p, sl]
                al = jnp.where(al > 0.0, al, NEG_SLOPE * al)
                w_v[p, sl] = jnp.exp(jnp.minimum(al, 50.0))
            pltpu.make_async_copy(hflat_hbm.at[gsidx_v.at[p]], rows_v.at[p],
                                  semr[p]).wait()

            def scale_g(g, carry):
                gl = pl.multiple_of(g * _L, _L)
                wch = w_v[p, pl.ds(gl, _L)]
                for e in range(_L):
                    wv = jnp.full((_L,), wch[e])
                    for j in range(OUT_DIM // _L):
                        sl2 = pl.ds(j * _L, _L)
                        rows_v[p, gl + e, sl2] = rows_v[p, gl + e, sl2] * wv
                return carry

            lax.fori_loop(0, EPT // _L, scale_g, 0)
            pltpu.sync_copy(rows_v.at[p], acc_sh.at[didx_v.at[p]], add=True)
            pltpu.sync_copy(w_v.at[p], dacc_sh.at[didx_v.at[p]], add=True)

        NPAIR = NCH // 2

        def pair_body(pr, carry):
            issue_loads(2 * pr + 1, 1)
            compute_and_scatter(0)

            @pl.when(pr < NPAIR - 1)
            def _():
                issue_loads(2 * pr + 2, 0)

            compute_and_scatter(1)
            return carry

        issue_loads(0, 0)
        lax.fori_loop(0, NPAIR, pair_body, 0)
        plsc.subcore_barrier()

        pltpu.sync_copy(acc_sh.at[pl.ds(s * tile_rows, tile_rows)],
                        acc_out.at[head, pl.ds(s * tile_rows, tile_rows)])
        pltpu.sync_copy(dacc_sh.at[pl.ds(s * tile_rows, tile_rows)],
                        dacc_out.at[head, pl.ds(s * tile_rows, tile_rows)])


def _epi_body(acc_ref, dacc_ref, bias_ref, out_ref):
    a = acc_ref[...]
    d = dacc_ref[...]
    r = a / (d + 1e-16)
    out_ref[...] = jnp.mean(r, axis=0) + bias_ref[...]


def _epilogue(acc, dacc, bias_g):
    grid = (N_NODES // BLK,)
    return pl.pallas_call(
        _epi_body,
        grid=grid,
        in_specs=[
            pl.BlockSpec((HEADS, BLK, OUT_DIM), lambda i: (0, i, 0)),
            pl.BlockSpec((HEADS, BLK, 1), lambda i: (0, i, 0)),
            pl.BlockSpec((1, OUT_DIM), lambda i: (0, 0)),
        ],
        out_specs=pl.BlockSpec((BLK, OUT_DIM), lambda i: (i, 0)),
        out_shape=jax.ShapeDtypeStruct((N_NODES, OUT_DIM), jnp.float32),
    )(acc, dacc, bias_g.reshape(1, -1))


def kernel(z, W1, b1, W2, b2, Wg, att_src, att_dst, bias_g, edge_index):
    N = z.shape[0]
    h_flat, a_src, a_dst = _dense_stage(z, W1, b1, W2, b2, Wg, att_src, att_dst)

    h_heads = jnp.pad(h_flat.reshape(N, HEADS, OUT_DIM).transpose(1, 0, 2),
                      ((0, 0), (0, N_PAD - N), (0, 0)))
    hflat = h_heads.reshape(HEADS * N_PAD, OUT_DIM)
    asrc_f = jnp.pad(a_src.T, ((0, 0), (0, N_PAD - N))).reshape(HEADS * N_PAD)
    adst_f = jnp.pad(a_dst.T, ((0, 0), (0, N_PAD - N))).reshape(HEADS * N_PAD)

    loops = jnp.arange(N, dtype=edge_index.dtype)
    npad_e = E_PAD - edge_index.shape[1] - N
    pad_idx = N + 16 + (jnp.arange(npad_e, dtype=jnp.int32) % 128)
    srcs = jnp.concatenate([edge_index[0], loops, pad_idx])
    dsts = jnp.concatenate([edge_index[1], loops, pad_idx])

    acc, dacc = _edge_kernel(hflat, asrc_f, adst_f, srcs, dsts)
    return _epilogue(acc[:, :N, :], dacc[:, :N].reshape(HEADS, N, 1), bias_g)


# speedup vs baseline: 33.6333x; 1.0222x over previous
import functools

import jax
import jax.numpy as jnp
from jax import lax
from jax.experimental import pallas as pl
from jax.experimental.pallas import tpu as pltpu, tpu_sc as plsc

HEADS = 4
OUT_DIM = 128
NEG_SLOPE = 0.2
BLK = 1000

N_NODES = 10000
N_PAD = 10240
EPT = 128
NCH = 162
E_PAD = 16 * NCH * EPT

_NC, _NS, _L = 2, 16, 16


def _dense_body(z_ref, W1_ref, b1_ref, W2_ref, b2_ref, Wg_ref, asv_ref, adv_ref,
                h_ref, asrc_ref, adst_ref):
    z = z_ref[...]
    x = jnp.maximum(z @ W1_ref[...] + b1_ref[...], 0.0)
    x = jnp.maximum(x @ W2_ref[...] + b2_ref[...], 0.0)
    h = x @ Wg_ref[...]
    h_ref[...] = h
    hh = h.reshape(h.shape[0], HEADS, OUT_DIM)
    asrc_ref[...] = jnp.sum(hh * asv_ref[...], axis=-1)
    adst_ref[...] = jnp.sum(hh * adv_ref[...], axis=-1)


def _dense_stage(z, W1, b1, W2, b2, Wg, att_src, att_dst):
    N = z.shape[0]
    grid = (N // BLK,)
    out_shapes = (
        jax.ShapeDtypeStruct((N, HEADS * OUT_DIM), jnp.float32),
        jax.ShapeDtypeStruct((N, HEADS), jnp.float32),
        jax.ShapeDtypeStruct((N, HEADS), jnp.float32),
    )
    full = lambda shape: pl.BlockSpec(shape, lambda i: tuple(0 for _ in shape))
    return pl.pallas_call(
        _dense_body,
        grid=grid,
        in_specs=[
            pl.BlockSpec((BLK, 128), lambda i: (i, 0)),
            full((128, 64)), full((1, 64)), full((64, 128)), full((1, 128)),
            full((128, HEADS * OUT_DIM)),
            full((1, HEADS, OUT_DIM)), full((1, HEADS, OUT_DIM)),
        ],
        out_specs=(
            pl.BlockSpec((BLK, HEADS * OUT_DIM), lambda i: (i, 0)),
            pl.BlockSpec((BLK, HEADS), lambda i: (i, 0)),
            pl.BlockSpec((BLK, HEADS), lambda i: (i, 0)),
        ),
        out_shape=out_shapes,
    )(z, W1, b1.reshape(1, -1), W2, b2.reshape(1, -1), Wg, att_src, att_dst)


_sc_mesh = plsc.VectorSubcoreMesh(core_axis_name="c", subcore_axis_name="s")


@functools.partial(
    pl.kernel, mesh=_sc_mesh,
    out_type=(
        jax.ShapeDtypeStruct((HEADS, N_PAD, OUT_DIM), jnp.float32),
        jax.ShapeDtypeStruct((HEADS, N_PAD), jnp.float32),
    ),
    scratch_types=[
        pltpu.VMEM((2, EPT), jnp.int32),
        pltpu.VMEM((2, EPT), jnp.int32),
        pltpu.VMEM((2, EPT), jnp.int32),
        pltpu.VMEM((2, EPT), jnp.int32),
        pltpu.VMEM((2, EPT), jnp.float32),
        pltpu.VMEM((2, EPT), jnp.float32),
        pltpu.VMEM((2, EPT), jnp.float32),
        pltpu.VMEM((2, EPT, OUT_DIM), jnp.float32),
        pltpu.VMEM((16, OUT_DIM), jnp.float32),
        pltpu.VMEM((N_PAD // _NS,), jnp.float32),
        pltpu.VMEM_SHARED((N_PAD, OUT_DIM), jnp.float32),
        pltpu.VMEM_SHARED((N_PAD,), jnp.float32),
        pltpu.SemaphoreType.DMA,
        pltpu.SemaphoreType.DMA,
        pltpu.SemaphoreType.DMA,
        pltpu.SemaphoreType.DMA,
        pltpu.SemaphoreType.DMA,
        pltpu.SemaphoreType.DMA,
        pltpu.SemaphoreType.DMA,
        pltpu.SemaphoreType.DMA,
    ],
)
def _edge_kernel(hflat_hbm, asrc_hbm, adst_hbm, srcs_hbm, dsts_hbm,
                 acc_out, dacc_out,
                 sidx_v, didx_v, gsidx_v, gdidx_v, asv_v, adv_v, w_v, rows_v,
                 zb_v, zd_v, acc_sh, dacc_sh,
                 semr0, semr1, sema0, sema1, semb0, semb1, sems0, sems1):
    c = lax.axis_index("c")
    s = lax.axis_index("s")
    tile_rows = N_PAD // _NS
    semr = (semr0, semr1)
    sema = (sema0, sema1)
    semb = (semb0, semb1)
    sems = (sems0, sems1)

    z16 = jnp.zeros((_L,), jnp.float32)
    for r in range(16):
        for j in range(OUT_DIM // _L):
            zb_v[r, pl.ds(j * _L, _L)] = z16
    for j in range(tile_rows // _L):
        zd_v[pl.ds(j * _L, _L)] = z16

    for hp in range(2):
        head = c * 2 + hp
        for r8 in range(tile_rows // 16):
            pltpu.sync_copy(zb_v, acc_sh.at[pl.ds(s * tile_rows + r8 * 16, 16)])
        pltpu.sync_copy(zd_v, dacc_sh.at[pl.ds(s * tile_rows, tile_rows)])
        plsc.subcore_barrier()

        off = head * N_PAD

        def issue_loads(i, p, wait_guard):
            if wait_guard is not None:
                drain = lambda: pltpu.make_async_copy(
                    rows_v.at[p], acc_sh.at[didx_v.at[p]], sems[p]).wait()
                if wait_guard is True:
                    drain()
                else:
                    pl.when(wait_guard)(drain)
            base = (s * NCH + i) * EPT
            pltpu.sync_copy(srcs_hbm.at[pl.ds(base, EPT)], sidx_v.at[p])
            pltpu.sync_copy(dsts_hbm.at[pl.ds(base, EPT)], didx_v.at[p])
            for g in range(EPT // _L):
                sl = pl.ds(g * _L, _L)
                gsidx_v[p, sl] = sidx_v[p, sl] + off
                gdidx_v[p, sl] = didx_v[p, sl] + off
            pltpu.async_copy(hflat_hbm.at[gsidx_v.at[p]], rows_v.at[p], semr[p])
            pltpu.async_copy(asrc_hbm.at[gsidx_v.at[p]], asv_v.at[p], sema[p])
            pltpu.async_copy(adst_hbm.at[gdidx_v.at[p]], adv_v.at[p], semb[p])

        def compute_and_scatter(p):
            pltpu.make_async_copy(asrc_hbm.at[gsidx_v.at[p]], asv_v.at[p],
                                  sema[p]).wait()
            pltpu.make_async_copy(adst_hbm.at[gdidx_v.at[p]], adv_v.at[p],
                                  semb[p]).wait()
            for g in range(EPT // _L):
                sl = pl.ds(g * _L, _L)
                al = asv_v[p, sl] + adv_v[p, sl]
                al = jnp.where(al > 0.0, al, NEG_SLOPE * al)
                w_v[p, sl] = jnp.exp(jnp.minimum(al, 50.0))
            pltpu.make_async_copy(hflat_hbm.at[gsidx_v.at[p]], rows_v.at[p],
                                  semr[p]).wait()

            def scale_g(g, carry):
                gl = pl.multiple_of(g * _L, _L)
                wch = w_v[p, pl.ds(gl, _L)]
                for e in range(_L):
                    wv = jnp.full((_L,), wch[e])
                    for j in range(OUT_DIM // _L):
                        sl2 = pl.ds(j * _L, _L)
                        rows_v[p, gl + e, sl2] = rows_v[p, gl + e, sl2] * wv
                return carry

            lax.fori_loop(0, EPT // _L, scale_g, 0)
            pltpu.async_copy(rows_v.at[p], acc_sh.at[didx_v.at[p]], sems[p],
                             add=True)
            pltpu.sync_copy(w_v.at[p], dacc_sh.at[didx_v.at[p]], add=True)

        NPAIR = NCH // 2

        def pair_body(pr, carry):
            issue_loads(2 * pr + 1, 1, pr > 0)
            compute_and_scatter(0)

            @pl.when(pr < NPAIR - 1)
            def _():
                issue_loads(2 * pr + 2, 0, True)

            compute_and_scatter(1)
            return carry

        issue_loads(0, 0, None)
        lax.fori_loop(0, NPAIR, pair_body, 0)
        pltpu.make_async_copy(rows_v.at[0], acc_sh.at[didx_v.at[0]],
                              sems[0]).wait()
        pltpu.make_async_copy(rows_v.at[1], acc_sh.at[didx_v.at[1]],
                              sems[1]).wait()
        plsc.subcore_barrier()

        pltpu.sync_copy(acc_sh.at[pl.ds(s * tile_rows, tile_rows)],
                        acc_out.at[head, pl.ds(s * tile_rows, tile_rows)])
        pltpu.sync_copy(dacc_sh.at[pl.ds(s * tile_rows, tile_rows)],
                        dacc_out.at[head, pl.ds(s * tile_rows, tile_rows)])


def _epi_body(acc_ref, dacc_ref, bias_ref, out_ref):
    a = acc_ref[...]
    d = dacc_ref[...]
    r = a / (d + 1e-16)
    out_ref[...] = jnp.mean(r, axis=0) + bias_ref[...]


def _epilogue(acc, dacc, bias_g):
    grid = (N_NODES // BLK,)
    return pl.pallas_call(
        _epi_body,
        grid=grid,
        in_specs=[
            pl.BlockSpec((HEADS, BLK, OUT_DIM), lambda i: (0, i, 0)),
            pl.BlockSpec((HEADS, BLK, 1), lambda i: (0, i, 0)),
            pl.BlockSpec((1, OUT_DIM), lambda i: (0, 0)),
        ],
        out_specs=pl.BlockSpec((BLK, OUT_DIM), lambda i: (i, 0)),
        out_shape=jax.ShapeDtypeStruct((N_NODES, OUT_DIM), jnp.float32),
    )(acc, dacc, bias_g.reshape(1, -1))


def kernel(z, W1, b1, W2, b2, Wg, att_src, att_dst, bias_g, edge_index):
    N = z.shape[0]
    h_flat, a_src, a_dst = _dense_stage(z, W1, b1, W2, b2, Wg, att_src, att_dst)

    h_heads = jnp.pad(h_flat.reshape(N, HEADS, OUT_DIM).transpose(1, 0, 2),
                      ((0, 0), (0, N_PAD - N), (0, 0)))
    hflat = h_heads.reshape(HEADS * N_PAD, OUT_DIM)
    asrc_f = jnp.pad(a_src.T, ((0, 0), (0, N_PAD - N))).reshape(HEADS * N_PAD)
    adst_f = jnp.pad(a_dst.T, ((0, 0), (0, N_PAD - N))).reshape(HEADS * N_PAD)

    loops = jnp.arange(N, dtype=edge_index.dtype)
    npad_e = E_PAD - edge_index.shape[1] - N
    pad_idx = N + 16 + (jnp.arange(npad_e, dtype=jnp.int32) % 128)
    srcs = jnp.concatenate([edge_index[0], loops, pad_idx])
    dsts = jnp.concatenate([edge_index[1], loops, pad_idx])

    acc, dacc = _edge_kernel(hflat, asrc_f, adst_f, srcs, dsts)
    return _epilogue(acc[:, :N, :], dacc[:, :N].reshape(HEADS, N, 1), bias_g)


# speedup vs baseline: 33.6406x; 1.0002x over previous
import functools

import jax
import jax.numpy as jnp
from jax import lax
from jax.experimental import pallas as pl
from jax.experimental.pallas import tpu as pltpu, tpu_sc as plsc

HEADS = 4
OUT_DIM = 128
NEG_SLOPE = 0.2
BLK = 1000

N_NODES = 10000
N_PAD = 10240
EPT = 128
NCH = 162
E_PAD = 16 * NCH * EPT

_NC, _NS, _L = 2, 16, 16


def _dense_body(z_ref, W1_ref, b1_ref, W2_ref, b2_ref, Wg_ref, asv_ref, adv_ref,
                h_ref, asrc_ref, adst_ref):
    z = z_ref[...]
    x = jnp.maximum(z @ W1_ref[...] + b1_ref[...], 0.0)
    x = jnp.maximum(x @ W2_ref[...] + b2_ref[...], 0.0)
    h = x @ Wg_ref[...]
    h_ref[...] = h
    hh = h.reshape(h.shape[0], HEADS, OUT_DIM)
    asrc_ref[...] = jnp.sum(hh * asv_ref[...], axis=-1)
    adst_ref[...] = jnp.sum(hh * adv_ref[...], axis=-1)


def _dense_stage(z, W1, b1, W2, b2, Wg, att_src, att_dst):
    N = z.shape[0]
    grid = (N // BLK,)
    out_shapes = (
        jax.ShapeDtypeStruct((N, HEADS * OUT_DIM), jnp.float32),
        jax.ShapeDtypeStruct((N, HEADS), jnp.float32),
        jax.ShapeDtypeStruct((N, HEADS), jnp.float32),
    )
    full = lambda shape: pl.BlockSpec(shape, lambda i: tuple(0 for _ in shape))
    return pl.pallas_call(
        _dense_body,
        grid=grid,
        in_specs=[
            pl.BlockSpec((BLK, 128), lambda i: (i, 0)),
            full((128, 64)), full((1, 64)), full((64, 128)), full((1, 128)),
            full((128, HEADS * OUT_DIM)),
            full((1, HEADS, OUT_DIM)), full((1, HEADS, OUT_DIM)),
        ],
        out_specs=(
            pl.BlockSpec((BLK, HEADS * OUT_DIM), lambda i: (i, 0)),
            pl.BlockSpec((BLK, HEADS), lambda i: (i, 0)),
            pl.BlockSpec((BLK, HEADS), lambda i: (i, 0)),
        ),
        out_shape=out_shapes,
    )(z, W1, b1.reshape(1, -1), W2, b2.reshape(1, -1), Wg, att_src, att_dst)


_edge_kernel_cache = {}


def _get_edge_kernel():
    if "k" not in _edge_kernel_cache:
        _edge_kernel_cache["k"] = _build_edge_kernel()
    return _edge_kernel_cache["k"]


def _build_edge_kernel():
  mesh = plsc.VectorSubcoreMesh(core_axis_name="c", subcore_axis_name="s")
  return pl.kernel(
    _edge_kernel_body, mesh=mesh,
    out_type=(
        jax.ShapeDtypeStruct((HEADS, N_PAD, OUT_DIM), jnp.float32),
        jax.ShapeDtypeStruct((HEADS, N_PAD), jnp.float32),
    ),
    scratch_types=[
        pltpu.VMEM((2, EPT), jnp.int32),
        pltpu.VMEM((2, EPT), jnp.int32),
        pltpu.VMEM((2, EPT), jnp.int32),
        pltpu.VMEM((2, EPT), jnp.int32),
        pltpu.VMEM((2, EPT), jnp.float32),
        pltpu.VMEM((2, EPT), jnp.float32),
        pltpu.VMEM((2, EPT), jnp.float32),
        pltpu.VMEM((2, EPT, OUT_DIM), jnp.float32),
        pltpu.VMEM((16, OUT_DIM), jnp.float32),
        pltpu.VMEM((N_PAD // _NS,), jnp.float32),
        pltpu.VMEM_SHARED((N_PAD, OUT_DIM), jnp.float32),
        pltpu.VMEM_SHARED((N_PAD,), jnp.float32),
        pltpu.SemaphoreType.DMA,
        pltpu.SemaphoreType.DMA,
        pltpu.SemaphoreType.DMA,
        pltpu.SemaphoreType.DMA,
        pltpu.SemaphoreType.DMA,
        pltpu.SemaphoreType.DMA,
        pltpu.SemaphoreType.DMA,
        pltpu.SemaphoreType.DMA,
    ],
  )


def _edge_kernel_body(hflat_hbm, asrc_hbm, adst_hbm, srcs_hbm, dsts_hbm,
                 acc_out, dacc_out,
                 sidx_v, didx_v, gsidx_v, gdidx_v, asv_v, adv_v, w_v, rows_v,
                 zb_v, zd_v, acc_sh, dacc_sh,
                 semr0, semr1, sema0, sema1, semb0, semb1, sems0, sems1):
    c = lax.axis_index("c")
    s = lax.axis_index("s")
    tile_rows = N_PAD // _NS
    semr = (semr0, semr1)
    sema = (sema0, sema1)
    semb = (semb0, semb1)
    sems = (sems0, sems1)

    z16 = jnp.zeros((_L,), jnp.float32)
    for r in range(16):
        for j in range(OUT_DIM // _L):
            zb_v[r, pl.ds(j * _L, _L)] = z16
    for j in range(tile_rows // _L):
        zd_v[pl.ds(j * _L, _L)] = z16

    for hp in range(2):
        head = c * 2 + hp
        for r8 in range(tile_rows // 16):
            pltpu.sync_copy(zb_v, acc_sh.at[pl.ds(s * tile_rows + r8 * 16, 16)])
        pltpu.sync_copy(zd_v, dacc_sh.at[pl.ds(s * tile_rows, tile_rows)])
        plsc.subcore_barrier()

        off = head * N_PAD

        def issue_loads(i, p, wait_guard):
            if wait_guard is not None:
                drain = lambda: pltpu.make_async_copy(
                    rows_v.at[p], acc_sh.at[didx_v.at[p]], sems[p]).wait()
                if wait_guard is True:
                    drain()
                else:
                    pl.when(wait_guard)(drain)
            base = (s * NCH + i) * EPT
            pltpu.sync_copy(srcs_hbm.at[pl.ds(base, EPT)], sidx_v.at[p])
            pltpu.sync_copy(dsts_hbm.at[pl.ds(base, EPT)], didx_v.at[p])
            for g in range(EPT // _L):
                sl = pl.ds(g * _L, _L)
                gsidx_v[p, sl] = sidx_v[p, sl] + off
                gdidx_v[p, sl] = didx_v[p, sl] + off
            pltpu.async_copy(hflat_hbm.at[gsidx_v.at[p]], rows_v.at[p], semr[p])
            pltpu.async_copy(asrc_hbm.at[gsidx_v.at[p]], asv_v.at[p], sema[p])
            pltpu.async_copy(adst_hbm.at[gdidx_v.at[p]], adv_v.at[p], semb[p])

        def compute_and_scatter(p):
            pltpu.make_async_copy(asrc_hbm.at[gsidx_v.at[p]], asv_v.at[p],
                                  sema[p]).wait()
            pltpu.make_async_copy(adst_hbm.at[gdidx_v.at[p]], adv_v.at[p],
                                  semb[p]).wait()
            for g in range(EPT // _L):
                sl = pl.ds(g * _L, _L)
                al = asv_v[p, sl] + adv_v[p, sl]
                al = jnp.where(al > 0.0, al, NEG_SLOPE * al)
                w_v[p, sl] = jnp.exp(jnp.minimum(al, 50.0))
            pltpu.make_async_copy(hflat_hbm.at[gsidx_v.at[p]], rows_v.at[p],
                                  semr[p]).wait()

            def scale_g(g, carry):
                gl = pl.multiple_of(g * _L, _L)
                wch = w_v[p, pl.ds(gl, _L)]
                for e in range(_L):
                    wv = jnp.full((_L,), wch[e])
                    for j in range(OUT_DIM // _L):
                        sl2 = pl.ds(j * _L, _L)
                        rows_v[p, gl + e, sl2] = rows_v[p, gl + e, sl2] * wv
                return carry

            lax.fori_loop(0, EPT // _L, scale_g, 0)
            pltpu.async_copy(rows_v.at[p], acc_sh.at[didx_v.at[p]], sems[p],
                             add=True)
            pltpu.sync_copy(w_v.at[p], dacc_sh.at[didx_v.at[p]], add=True)

        NPAIR = NCH // 2

        def pair_body(pr, carry):
            issue_loads(2 * pr + 1, 1, pr > 0)
            compute_and_scatter(0)

            @pl.when(pr < NPAIR - 1)
            def _():
                issue_loads(2 * pr + 2, 0, True)

            compute_and_scatter(1)
            return carry

        issue_loads(0, 0, None)
        lax.fori_loop(0, NPAIR, pair_body, 0)
        pltpu.make_async_copy(rows_v.at[0], acc_sh.at[didx_v.at[0]],
                              sems[0]).wait()
        pltpu.make_async_copy(rows_v.at[1], acc_sh.at[didx_v.at[1]],
                              sems[1]).wait()
        plsc.subcore_barrier()

        pltpu.sync_copy(acc_sh.at[pl.ds(s * tile_rows, tile_rows)],
                        acc_out.at[head, pl.ds(s * tile_rows, tile_rows)])
        pltpu.sync_copy(dacc_sh.at[pl.ds(s * tile_rows, tile_rows)],
                        dacc_out.at[head, pl.ds(s * tile_rows, tile_rows)])


def _epi_body(acc_ref, dacc_ref, bias_ref, out_ref):
    a = acc_ref[...]
    d = dacc_ref[...]
    r = a / (d + 1e-16)
    out_ref[...] = jnp.mean(r, axis=0) + bias_ref[...]


def _epilogue(acc, dacc, bias_g):
    grid = (N_NODES // BLK,)
    return pl.pallas_call(
        _epi_body,
        grid=grid,
        in_specs=[
            pl.BlockSpec((HEADS, BLK, OUT_DIM), lambda i: (0, i, 0)),
            pl.BlockSpec((HEADS, BLK, 1), lambda i: (0, i, 0)),
            pl.BlockSpec((1, OUT_DIM), lambda i: (0, 0)),
        ],
        out_specs=pl.BlockSpec((BLK, OUT_DIM), lambda i: (i, 0)),
        out_shape=jax.ShapeDtypeStruct((N_NODES, OUT_DIM), jnp.float32),
    )(acc, dacc, bias_g.reshape(1, -1))


def kernel(z, W1, b1, W2, b2, Wg, att_src, att_dst, bias_g, edge_index):
    N = z.shape[0]
    h_flat, a_src, a_dst = _dense_stage(z, W1, b1, W2, b2, Wg, att_src, att_dst)

    h_heads = jnp.pad(h_flat.reshape(N, HEADS, OUT_DIM).transpose(1, 0, 2),
                      ((0, 0), (0, N_PAD - N), (0, 0)))
    hflat = h_heads.reshape(HEADS * N_PAD, OUT_DIM)
    asrc_f = jnp.pad(a_src.T, ((0, 0), (0, N_PAD - N))).reshape(HEADS * N_PAD)
    adst_f = jnp.pad(a_dst.T, ((0, 0), (0, N_PAD - N))).reshape(HEADS * N_PAD)

    loops = jnp.arange(N, dtype=edge_index.dtype)
    npad_e = E_PAD - edge_index.shape[1] - N
    pad_idx = N + 16 + (jnp.arange(npad_e, dtype=jnp.int32) % 128)
    srcs = jnp.concatenate([edge_index[0], loops, pad_idx])
    dsts = jnp.concatenate([edge_index[1], loops, pad_idx])

    acc, dacc = _get_edge_kernel()(hflat, asrc_f, adst_f, srcs, dsts)
    return _epilogue(acc[:, :N, :], dacc[:, :N].reshape(HEADS, N, 1), bias_g)
